# Initial kernel scaffold; baseline (speedup 1.0000x reference)
import jax
import jax.numpy as jnp
from jax.experimental import pallas as pl


def kernel(x, idx, W2, b2):
    raise NotImplementedError("write your pallas kernel here")


# speedup vs baseline: 1.1412x; 1.1412x over previous
import functools

import jax
import jax.numpy as jnp
from jax import lax
from jax.experimental import pallas as pl
from jax.experimental.pallas import tpu as pltpu
from jax.experimental.pallas import tpu_sc as plsc

N = 100000
D = 128

NC = 2
NS = 16
NW = NC * NS

CHUNK = 128
NP = 102400
ROWS_PER_W = NP // NW
NCHUNK = ROWS_PER_W // CHUNK


def _sc_gather(x, idx3):
    mesh = plsc.VectorSubcoreMesh(
        core_axis_name="c", subcore_axis_name="s", num_cores=NC, num_subcores=NS
    )

    @functools.partial(
        pl.kernel,
        out_type=jax.ShapeDtypeStruct((NP, D), jnp.float32),
        mesh=mesh,
        scratch_types=[
            pltpu.VMEM((NCHUNK, CHUNK), jnp.int32),
            pltpu.VMEM((CHUNK, D), jnp.float32),
            pltpu.SemaphoreType.DMA,
        ],
    )
    def gather_kernel(x_hbm, idx_hbm, g_hbm, idx_v, buf, sem):
        wid = lax.axis_index("s") * NC + lax.axis_index("c")
        pltpu.sync_copy(idx_hbm.at[wid], idx_v)
        base = wid * ROWS_PER_W

        def chunk_body(i, carry):
            pltpu.async_copy(x_hbm.at[idx_v.at[i]], buf, sem).wait()
            pltpu.sync_copy(buf, g_hbm.at[pl.ds(base + i * CHUNK, CHUNK)])
            return carry

        lax.fori_loop(0, NCHUNK, chunk_body, 0, unroll=False)

    return gather_kernel(x, idx3)


def _tc_linear(g, x, wa, wb, b2):
    R = 2000
    nblk = N // R

    def body(g_ref, x_ref, wa_ref, wb_ref, b_ref, o_ref):
        acc = jnp.dot(g_ref[...], wa_ref[...], preferred_element_type=jnp.float32)
        acc = acc + jnp.dot(x_ref[...], wb_ref[...], preferred_element_type=jnp.float32)
        o_ref[...] = acc + b_ref[...]

    return pl.pallas_call(
        body,
        grid=(nblk,),
        in_specs=[
            pl.BlockSpec((R, D), lambda i: (i, 0)),
            pl.BlockSpec((R, D), lambda i: (i, 0)),
            pl.BlockSpec((D, D), lambda i: (0, 0)),
            pl.BlockSpec((D, D), lambda i: (0, 0)),
            pl.BlockSpec((1, D), lambda i: (0, 0)),
        ],
        out_specs=pl.BlockSpec((R, D), lambda i: (i, 0)),
        out_shape=jax.ShapeDtypeStruct((N, D), jnp.float32),
        compiler_params=pltpu.CompilerParams(
            dimension_semantics=("arbitrary",),
        ),
    )(g, x, wa, wb, b2)


def kernel(x, idx, W2, b2):
    idx_pad = jnp.concatenate([idx, jnp.zeros((NP - N,), jnp.int32)])
    idx3 = idx_pad.reshape(NW, NCHUNK, CHUNK)
    g = _sc_gather(x, idx3)
    wa = W2[:D]
    wb = W2[D:]
    return _tc_linear(g, x, wa, wb, b2.reshape(1, D))


# speedup vs baseline: 1.3052x; 1.1438x over previous
import functools

import jax
import jax.numpy as jnp
from jax import lax
from jax.experimental import pallas as pl
from jax.experimental.pallas import tpu as pltpu
from jax.experimental.pallas import tpu_sc as plsc

N = 100000
D = 128

NC = 2
NS = 16
NW = NC * NS

CHUNK = 128
NP = 102400
ROWS_PER_W = NP // NW
NCHUNK = ROWS_PER_W // CHUNK


def _sc_gather(x, idx3):
    mesh = plsc.VectorSubcoreMesh(
        core_axis_name="c", subcore_axis_name="s", num_cores=NC, num_subcores=NS
    )

    NBUF = 4

    @functools.partial(
        pl.kernel,
        out_type=jax.ShapeDtypeStruct((NP, D), jnp.float32),
        mesh=mesh,
        scratch_types=[
            pltpu.VMEM((NCHUNK, CHUNK), jnp.int32),
            pltpu.VMEM((NBUF, CHUNK, D), jnp.float32),
            pltpu.SemaphoreType.DMA((NBUF,)),
        ],
    )
    def gather_kernel(x_hbm, idx_hbm, g_hbm, idx_v, buf, gsem):
        wid = lax.axis_index("s") * NC + lax.axis_index("c")
        pltpu.sync_copy(idx_hbm.at[wid], idx_v)
        base = wid * ROWS_PER_W

        for j in range(NBUF - 1):
            pltpu.async_copy(x_hbm.at[idx_v.at[j]], buf.at[j], gsem.at[j])

        def chunk_body(i, carry):
            slot = lax.rem(i, NBUF)
            nxt = i + NBUF - 1
            nslot = lax.rem(nxt, NBUF)

            @pl.when(nxt < NCHUNK)
            def _():
                pltpu.async_copy(x_hbm.at[idx_v.at[nxt]], buf.at[nslot], gsem.at[nslot])

            pltpu.make_async_copy(
                x_hbm.at[idx_v.at[i]], buf.at[slot], gsem.at[slot]
            ).wait()
            pltpu.sync_copy(buf.at[slot], g_hbm.at[pl.ds(base + i * CHUNK, CHUNK)])
            return carry

        lax.fori_loop(0, NCHUNK, chunk_body, 0, unroll=False)

    return gather_kernel(x, idx3)


def _tc_linear(g, x, wa, wb, b2):
    R = 2000
    nblk = N // R

    def body(g_ref, x_ref, wa_ref, wb_ref, b_ref, o_ref):
        acc = jnp.dot(g_ref[...], wa_ref[...], preferred_element_type=jnp.float32)
        acc = acc + jnp.dot(x_ref[...], wb_ref[...], preferred_element_type=jnp.float32)
        o_ref[...] = acc + b_ref[...]

    return pl.pallas_call(
        body,
        grid=(nblk,),
        in_specs=[
            pl.BlockSpec((R, D), lambda i: (i, 0)),
            pl.BlockSpec((R, D), lambda i: (i, 0)),
            pl.BlockSpec((D, D), lambda i: (0, 0)),
            pl.BlockSpec((D, D), lambda i: (0, 0)),
            pl.BlockSpec((1, D), lambda i: (0, 0)),
        ],
        out_specs=pl.BlockSpec((R, D), lambda i: (i, 0)),
        out_shape=jax.ShapeDtypeStruct((N, D), jnp.float32),
        compiler_params=pltpu.CompilerParams(
            dimension_semantics=("arbitrary",),
        ),
    )(g, x, wa, wb, b2)


def kernel(x, idx, W2, b2):
    idx_pad = jnp.concatenate([idx, jnp.zeros((NP - N,), jnp.int32)])
    idx3 = idx_pad.reshape(NW, NCHUNK, CHUNK)
    g = _sc_gather(x, idx3)
    wa = W2[:D]
    wb = W2[D:]
    return _tc_linear(g, x, wa, wb, b2.reshape(1, D))
